# Initial kernel scaffold; baseline (speedup 1.0000x reference)
import jax
import jax.numpy as jnp
from jax.experimental import pallas as pl


def kernel(indices, table):
    raise NotImplementedError("write your pallas kernel here")


# speedup vs baseline: 1.7945x; 1.7945x over previous
import functools

import jax
import jax.numpy as jnp
from jax import lax
from jax.experimental import pallas as pl
from jax.experimental.pallas import tpu as pltpu
from jax.experimental.pallas import tpu_sc as plsc

NC = 2
NS = 16
NW = NC * NS

VOCAB = 1_000_000
D = 64
B_TOTAL = 16384 * 50
B_PER_W = B_TOTAL // NW
CHUNK = 512
KSUB = CHUNK // 128
NCHUNK = B_PER_W // CHUNK


def _sc_gather(table, idx2d):
    mesh = plsc.VectorSubcoreMesh(
        core_axis_name="c", subcore_axis_name="s", num_cores=NC, num_subcores=NS
    )

    @functools.partial(
        pl.kernel,
        out_type=jax.ShapeDtypeStruct((B_TOTAL, D), jnp.float32),
        mesh=mesh,
        scratch_types=[
            pltpu.VMEM((KSUB, 128), jnp.int32),
            pltpu.VMEM((CHUNK, D), jnp.float32),
            pltpu.SemaphoreType.DMA,
        ],
        compiler_params=pltpu.CompilerParams(use_tc_tiling_on_sc=False),
    )
    def k(table_hbm, idx_hbm, out_hbm, idx_v, rows_v, sem):
        wid = lax.axis_index("s") * NC + lax.axis_index("c")

        def body(g, carry):
            row0 = wid * (B_PER_W // 128) + g * KSUB
            pltpu.sync_copy(idx_hbm.at[pl.ds(row0, KSUB)], idx_v)
            copies = [
                pltpu.async_copy(
                    table_hbm.at[idx_v.at[j]],
                    rows_v.at[pl.ds(j * 128, 128)],
                    sem,
                )
                for j in range(KSUB)
            ]
            for cp in copies:
                cp.wait()
            base = wid * B_PER_W + g * CHUNK
            pltpu.sync_copy(rows_v, out_hbm.at[pl.ds(base, CHUNK)])
            return carry

        lax.fori_loop(0, NCHUNK, body, 0)

    return k(table, idx2d)


def kernel(indices, table):
    idx2d = indices.astype(jnp.int32).reshape(B_TOTAL // 128, 128)
    out = _sc_gather(table, idx2d)
    return out.reshape(indices.shape[0], indices.shape[1], D)


# speedup vs baseline: 1.8758x; 1.0453x over previous
import functools

import jax
import jax.numpy as jnp
from jax import lax
from jax.experimental import pallas as pl
from jax.experimental.pallas import tpu as pltpu
from jax.experimental.pallas import tpu_sc as plsc

NC = 2
NS = 16
NW = NC * NS

VOCAB = 1_000_000
D = 64
B_TOTAL = 16384 * 50
B_PER_W = B_TOTAL // NW
CHUNK = 512
KSUB = CHUNK // 128
NCHUNK = B_PER_W // CHUNK
NROW = NCHUNK * KSUB


def _sc_gather(table, idx2d):
    mesh = plsc.VectorSubcoreMesh(
        core_axis_name="c", subcore_axis_name="s", num_cores=NC, num_subcores=NS
    )

    @functools.partial(
        pl.kernel,
        out_type=jax.ShapeDtypeStruct((B_TOTAL, D), jnp.float32),
        mesh=mesh,
        scratch_types=[
            pltpu.VMEM((NROW, 128), jnp.int32),
            pltpu.VMEM((CHUNK, D), jnp.float32),
            pltpu.VMEM((CHUNK, D), jnp.float32),
            pltpu.SemaphoreType.DMA,
            pltpu.SemaphoreType.DMA,
            pltpu.SemaphoreType.DMA,
            pltpu.SemaphoreType.DMA,
        ],
        compiler_params=pltpu.CompilerParams(use_tc_tiling_on_sc=False),
    )
    def k(table_hbm, idx_hbm, out_hbm, idx_v, rows0, rows1, g0, g1, w0, w1):
        wid = lax.axis_index("s") * NC + lax.axis_index("c")
        rows = (rows0, rows1)
        gsem = (g0, g1)
        wsem = (w0, w1)
        base = wid * B_PER_W

        pltpu.sync_copy(idx_hbm.at[pl.ds(wid * NROW, NROW)], idx_v)

        def start_gather(c, s):
            for j in range(KSUB):
                pltpu.async_copy(
                    table_hbm.at[idx_v.at[c * KSUB + j]],
                    rows[s].at[pl.ds(j * 128, 128)],
                    gsem[s],
                )

        def wait_gather(s):
            pltpu.make_async_copy(
                table_hbm.at[pl.ds(0, CHUNK)], rows[s], gsem[s]
            ).wait()

        def start_wb(c, s):
            pltpu.async_copy(
                rows[s], out_hbm.at[pl.ds(base + c * CHUNK, CHUNK)], wsem[s]
            )

        def wait_wb(s):
            pltpu.make_async_copy(
                rows[s], out_hbm.at[pl.ds(0, CHUNK)], wsem[s]
            ).wait()

        start_gather(0, 0)
        start_gather(1, 1)
        wait_gather(0)
        start_wb(0, 0)

        def body(i, carry):
            c = 1 + 2 * i
            wait_wb(0)
            start_gather(c + 1, 0)
            wait_gather(1)
            start_wb(c, 1)
            wait_wb(1)
            start_gather(c + 2, 1)
            wait_gather(0)
            start_wb(c + 1, 0)
            return carry

        lax.fori_loop(0, (NCHUNK - 2) // 2, body, 0)

        wait_wb(0)
        wait_gather(1)
        start_wb(NCHUNK - 1, 1)
        wait_wb(1)

    return k(table, idx2d)


def kernel(indices, table):
    idx2d = indices.astype(jnp.int32).reshape(B_TOTAL // 128, 128)
    out = _sc_gather(table, idx2d)
    return out.reshape(indices.shape[0], indices.shape[1], D)
